# baseline (speedup 1.0000x reference)
import functools

import jax
import jax.numpy as jnp
from jax import lax
from jax.experimental import pallas as pl
from jax.experimental.pallas import tpu as pltpu
from jax.experimental.pallas import tpu_sc as plsc

DIM = 64
NEG = 10
NC = 2
NS = 16
NW = NC * NS
LANES = 16
GRP = 8
CHUNK = 16


def _sc_scores(u_w, v_w3, u_pos, v_pos, v_neg_flat, batch):
    bpw = batch // NW
    nchunks = bpw // CHUNK
    mesh = plsc.VectorSubcoreMesh(
        core_axis_name="c", subcore_axis_name="s", num_cores=NC, num_subcores=NS
    )

    @functools.partial(
        pl.kernel,
        out_type=[
            jax.ShapeDtypeStruct((batch,), jnp.float32),
            jax.ShapeDtypeStruct((batch,), jnp.float32),
        ],
        mesh=mesh,
        compiler_params=pltpu.CompilerParams(needs_layout_passes=False),
        scratch_types=[
            pltpu.VMEM((batch // NW,), jnp.int32),
            pltpu.VMEM((batch // NW,), jnp.int32),
            pltpu.VMEM((NEG * batch // NW,), jnp.int32),
            pltpu.VMEM((CHUNK, GRP, DIM), jnp.float32),
            pltpu.VMEM((CHUNK, GRP, DIM), jnp.float32),
            pltpu.VMEM((NEG * CHUNK // 2, GRP, DIM), jnp.float32),
            pltpu.VMEM((batch // NW,), jnp.float32),
            pltpu.VMEM((batch // NW,), jnp.float32),
            pltpu.SemaphoreType.DMA,
        ],
    )
    def sc_kernel(u_w2, v_w, up, vp, vn, score_out, nscore_out,
                  idx_u, idx_v, idx_n,
                  rows_u, rows_v, rows_n, sc_chunk, nc_chunk, sem):
        wid = lax.axis_index("s") * NC + lax.axis_index("c")
        base = wid * bpw
        lane_iota = lax.iota(jnp.int32, LANES)

        pltpu.sync_copy(up.at[pl.ds(base, bpw)], idx_u)
        pltpu.sync_copy(vp.at[pl.ds(base, bpw)], idx_v)
        pltpu.sync_copy(vn.at[pl.ds(base * NEG, bpw * NEG)], idx_n)

        def chunk_body(c, _):
            boff = c * CHUNK
            csl = pl.ds(boff, CHUNK)
            qu = idx_u[csl] >> 3
            qv = idx_v[csl] >> 3
            for j in range(CHUNK):
                pltpu.async_copy(u_w2.at[qu[j]], rows_u.at[j], sem)
                pltpu.async_copy(v_w.at[qv[j]], rows_v.at[j], sem)

            nh = NEG // 2

            def issue_negs(h):
                for t in range(nh):
                    qn = plsc.load_gather(
                        idx_n,
                        [(boff + lane_iota) * NEG + (h * nh + t)]) >> 3
                    for j in range(LANES):
                        pltpu.async_copy(
                            v_w.at[qn[j]], rows_n.at[t * LANES + j], sem)

            def drain(k):
                for _ in range(k):
                    pltpu.make_async_copy(
                        v_w.at[pl.ds(0, CHUNK)],
                        rows_n.at[pl.ds(0, CHUNK)], sem).wait()

            ru = idx_u[csl] & 7
            rv = idx_v[csl] & 7

            def make_neg_meta(h):
                rows = []
                for n in range(nh):
                    ni = plsc.load_gather(
                        idx_n,
                        [(boff + lane_iota) * NEG + (h * nh + n)])
                    rows.append(ni & 7)
                return rows

            def dpass(h, acc_s, acc_n, with_uv, nrow):
                def dloop(d, carry):
                    a_s, a_n = carry
                    dv = jnp.full((LANES,), 0, jnp.int32) + d
                    gu = plsc.load_gather(rows_u, [lane_iota, ru, dv])
                    gn = None
                    for n in range(nh):
                        gx = plsc.load_gather(
                            rows_n, [n * LANES + lane_iota, nrow[n], dv])
                        gn = gx if gn is None else gn + gx
                    if with_uv:
                        gv = plsc.load_gather(rows_v, [lane_iota, rv, dv])
                        a_s = a_s + gu * gv
                    return (a_s, a_n + gu * gn)

                return lax.fori_loop(0, DIM, dloop, (acc_s, acc_n))

            zeros = jnp.zeros((LANES,), jnp.float32)
            issue_negs(0)
            nrow0 = make_neg_meta(0)
            drain(nh + 2)
            acc_s, acc_n = dpass(0, zeros, zeros, True, nrow0)
            issue_negs(1)
            nrow1 = make_neg_meta(1)
            drain(nh)
            acc_s, acc_n = dpass(1, acc_s, acc_n, False, nrow1)
            sc_chunk[csl] = acc_s
            nc_chunk[csl] = acc_n
            return 0

        lax.fori_loop(0, nchunks, chunk_body, 0)
        pltpu.sync_copy(sc_chunk, score_out.at[pl.ds(base, bpw)])
        pltpu.sync_copy(nc_chunk, nscore_out.at[pl.ds(base, bpw)])

    return sc_kernel(u_w, v_w3, u_pos, v_pos, v_neg_flat)


def _tc_loss_body(s_ref, n_ref, o_ref):
    s = s_ref[...]
    n = n_ref[...]
    val = jax.nn.log_sigmoid(s) + jax.nn.log_sigmoid(-n)
    o_ref[0, 0] = -jnp.sum(val)


def kernel(u_pos, v_pos, v_neg, batch_size, u_weight, v_weight):
    batch = u_pos.shape[0]
    vocab = v_weight.shape[0]
    scores, nscores = _sc_scores(
        u_weight.reshape(vocab // GRP, GRP, DIM),
        v_weight.reshape(vocab // GRP, GRP, DIM),
        u_pos.astype(jnp.int32),
        v_pos.astype(jnp.int32),
        v_neg.reshape(-1).astype(jnp.int32),
        batch,
    )
    rows = batch // 128
    loss_sum = pl.pallas_call(
        _tc_loss_body,
        out_shape=jax.ShapeDtypeStruct((1, 1), jnp.float32),
        out_specs=pl.BlockSpec(memory_space=pltpu.SMEM),
    )(scores.reshape(rows, 128), nscores.reshape(rows, 128))
    return loss_sum[0, 0] / batch_size


# speedup vs baseline: 1.0234x; 1.0234x over previous
import functools

import jax
import jax.numpy as jnp
from jax import lax
from jax.experimental import pallas as pl
from jax.experimental.pallas import tpu as pltpu
from jax.experimental.pallas import tpu_sc as plsc

DIM = 64
NEG = 10
NC = 2
NS = 16
NW = NC * NS
LANES = 16
GRP = 8
CHUNK = 16


def _sc_scores(u_w, v_w3, u_pos, v_pos, v_neg_flat, batch):
    bpw = batch // NW
    nchunks = bpw // CHUNK
    mesh = plsc.VectorSubcoreMesh(
        core_axis_name="c", subcore_axis_name="s", num_cores=NC, num_subcores=NS
    )

    @functools.partial(
        pl.kernel,
        out_type=[
            jax.ShapeDtypeStruct((batch,), jnp.float32),
            jax.ShapeDtypeStruct((batch,), jnp.float32),
        ],
        mesh=mesh,
        compiler_params=pltpu.CompilerParams(needs_layout_passes=False),
        scratch_types=[
            pltpu.VMEM((batch // NW,), jnp.int32),
            pltpu.VMEM((batch // NW,), jnp.int32),
            pltpu.VMEM((NEG * batch // NW,), jnp.int32),
            pltpu.VMEM((CHUNK, GRP, DIM), jnp.float32),
            pltpu.VMEM((CHUNK, GRP, DIM), jnp.float32),
            pltpu.VMEM((NEG * CHUNK // 2, GRP, DIM), jnp.float32),
            pltpu.VMEM((batch // NW,), jnp.float32),
            pltpu.VMEM((batch // NW,), jnp.float32),
            pltpu.SemaphoreType.DMA,
        ],
    )
    def sc_kernel(u_w2, v_w, up, vp, vn, score_out, nscore_out,
                  idx_u, idx_v, idx_n,
                  rows_u, rows_v, rows_n, sc_chunk, nc_chunk, sem):
        wid = lax.axis_index("s") * NC + lax.axis_index("c")
        base = wid * bpw
        lane_iota = lax.iota(jnp.int32, LANES)

        pltpu.sync_copy(up.at[pl.ds(base, bpw)], idx_u)
        pltpu.sync_copy(vp.at[pl.ds(base, bpw)], idx_v)
        pltpu.sync_copy(vn.at[pl.ds(base * NEG, bpw * NEG)], idx_n)

        def chunk_body(c, _):
            boff = c * CHUNK
            csl = pl.ds(boff, CHUNK)
            qu = idx_u[csl] >> 3
            qv = idx_v[csl] >> 3
            for j in range(CHUNK):
                pltpu.async_copy(u_w2.at[qu[j]], rows_u.at[j], sem)
                pltpu.async_copy(v_w.at[qv[j]], rows_v.at[j], sem)

            nh = NEG // 2

            def issue_negs(h):
                for t in range(nh):
                    qn = plsc.load_gather(
                        idx_n,
                        [(boff + lane_iota) * NEG + (h * nh + t)]) >> 3
                    for j in range(LANES):
                        pltpu.async_copy(
                            v_w.at[qn[j]], rows_n.at[t * LANES + j], sem)

            def drain(k):
                for _ in range(k):
                    pltpu.make_async_copy(
                        v_w.at[pl.ds(0, CHUNK)],
                        rows_n.at[pl.ds(0, CHUNK)], sem).wait()

            ru = idx_u[csl] & 7
            rv = idx_v[csl] & 7

            def neg_meta(h, n):
                ni = plsc.load_gather(
                    idx_n, [(boff + lane_iota) * NEG + (h * nh + n)])
                return ni & 7

            def uv_pass(acc_s):
                def dloop(d, a_s):
                    dv = jnp.full((LANES,), 0, jnp.int32) + d
                    gu = plsc.load_gather(rows_u, [lane_iota, ru, dv])
                    gv = plsc.load_gather(rows_v, [lane_iota, rv, dv])
                    return a_s + gu * gv
                return lax.fori_loop(0, DIM, dloop, acc_s)

            def n_pass(n, nrow, acc_n):
                def dloop(d, a_n):
                    dv = jnp.full((LANES,), 0, jnp.int32) + d
                    gu = plsc.load_gather(rows_u, [lane_iota, ru, dv])
                    gx = plsc.load_gather(
                        rows_n, [n * LANES + lane_iota, nrow, dv])
                    return a_n + gu * gx
                return lax.fori_loop(0, DIM, dloop, acc_n)

            zeros = jnp.zeros((LANES,), jnp.float32)
            issue_negs(0)
            metas0 = [neg_meta(0, n) for n in range(nh)]
            drain(3)
            acc_s = uv_pass(zeros)
            acc_n = n_pass(0, metas0[0], zeros)
            for n in range(1, nh):
                drain(1)
                acc_n = n_pass(n, metas0[n], acc_n)
            issue_negs(1)
            metas1 = [neg_meta(1, n) for n in range(nh)]
            for n in range(nh):
                drain(1)
                acc_n = n_pass(n, metas1[n], acc_n)
            sc_chunk[csl] = acc_s
            nc_chunk[csl] = acc_n
            return 0

        lax.fori_loop(0, nchunks, chunk_body, 0)
        pltpu.sync_copy(sc_chunk, score_out.at[pl.ds(base, bpw)])
        pltpu.sync_copy(nc_chunk, nscore_out.at[pl.ds(base, bpw)])

    return sc_kernel(u_w, v_w3, u_pos, v_pos, v_neg_flat)


def _tc_loss_body(s_ref, n_ref, o_ref):
    s = s_ref[...]
    n = n_ref[...]
    val = jax.nn.log_sigmoid(s) + jax.nn.log_sigmoid(-n)
    o_ref[0, 0] = -jnp.sum(val)


def kernel(u_pos, v_pos, v_neg, batch_size, u_weight, v_weight):
    batch = u_pos.shape[0]
    vocab = v_weight.shape[0]
    scores, nscores = _sc_scores(
        u_weight.reshape(vocab // GRP, GRP, DIM),
        v_weight.reshape(vocab // GRP, GRP, DIM),
        u_pos.astype(jnp.int32),
        v_pos.astype(jnp.int32),
        v_neg.reshape(-1).astype(jnp.int32),
        batch,
    )
    rows = batch // 128
    loss_sum = pl.pallas_call(
        _tc_loss_body,
        out_shape=jax.ShapeDtypeStruct((1, 1), jnp.float32),
        out_specs=pl.BlockSpec(memory_space=pltpu.SMEM),
    )(scores.reshape(rows, 128), nscores.reshape(rows, 128))
    return loss_sum[0, 0] / batch_size
